# baseline (device time: 54627 ns/iter reference)
import jax
import jax.numpy as jnp
from jax import lax
from jax.experimental import pallas as pl
from jax.experimental.pallas import tpu as pltpu

N_DEV = 8
M = 1024
N = 1024
K = 4096
KB = 8

SLAB_COLS = ((0, 384), (384, 768), (768, 1024))
SLAB_AXES = ((0, 1, 2), (1, 2, 0), (2, 0, 1))


def kernel(dy, W):
    def body(dy_ref, w_ref, out_ref, acc_bf, a_vm, b_vm, a_bf, b_bf, *bufs):
        in_sems = bufs[9]
        send_sems = bufs[10]
        recv_sems = bufs[11]

        me = lax.axis_index("i")
        zb = lax.shift_right_logical(me, 2) & 1
        yb = lax.shift_right_logical(me, 1) & 1
        xb = (me ^ lax.shift_right_logical(me, 1)) & 1

        p_z = me ^ 4
        p_y = (me & 4) | ((me & 3) ^ 3)
        p_x = (me & 4) | ((me & 3) ^ 1)
        bit = {0: zb, 1: yb, 2: xb}
        partner = {0: p_z, 1: p_y, 2: p_x}

        kc = K // KB

        def load(kb, slot):
            cpa = pltpu.make_async_copy(
                dy_ref.at[:, kb * kc:(kb + 1) * kc], a_vm.at[slot],
                in_sems.at[slot, 0])
            cpb = pltpu.make_async_copy(
                w_ref.at[:, kb * kc:(kb + 1) * kc], b_vm.at[slot],
                in_sems.at[slot, 1])
            cpa.start()
            cpb.start()
            return cpa, cpb

        pending = load(0, 0)
        for kb in range(KB):
            slot = kb % 2
            cpa, cpb = pending
            if kb + 1 < KB:
                pending = load(kb + 1, 1 - slot)
            cpa.wait()
            cpb.wait()
            a_bf[:, kb * kc:(kb + 1) * kc] = a_vm[slot].astype(jnp.bfloat16)
            b_bf[:, kb * kc:(kb + 1) * kc] = b_vm[slot].astype(jnp.bfloat16)

        barrier_sem = pltpu.get_barrier_semaphore()
        for nbr in (p_x, p_y, p_z):
            pl.semaphore_signal(
                barrier_sem, inc=1,
                device_id=(nbr,), device_id_type=pl.DeviceIdType.MESH,
            )
        pl.semaphore_wait(barrier_sem, 3)

        plans = []
        for s in range(3):
            a0, a1, a2 = SLAB_AXES[s]
            b0, b1, b2 = bit[a0], bit[a1], bit[a2]
            p0, p1, p2 = partner[a0], partner[a1], partner[a2]
            my_b1 = b0 * 512
            my_b2 = my_b1 + b1 * 256
            my_b3 = my_b2 + b2 * 128
            plans.append([
                ("rs", 512, p0, (1 - b0) * 512, my_b1),
                ("rs", 256, p1, my_b1 + (1 - b1) * 256, my_b2),
                ("rs", 128, p2, my_b2 + (1 - b2) * 128, my_b3),
                ("ag", 128, p2, my_b3, my_b3),
                ("ag", 256, p1, my_b2, my_b2),
                ("ag", 512, p0, my_b1, my_b1),
            ])

        def sub_dot(s, row_off):
            c0, c1 = SLAB_COLS[s]
            a = a_bf[pl.ds(row_off, 512), :]
            b = b_bf[c0:c1, :]
            p = lax.dot_general(
                a, b, (((1,), (1,)), ((), ())),
                preferred_element_type=jnp.float32,
            )
            acc_bf[pl.ds(row_off, 512), c0:c1] = p.astype(jnp.bfloat16)

        def issue(s, t):
            phase, rows, pid, send_off, _ = plans[s][t]
            c0, _ = SLAB_COLS[s]
            cw = SLAB_COLS[s][1] - c0
            if phase == "rs":
                src = acc_bf.at[pl.ds(send_off, rows), pl.ds(c0, cw)]
                dst = bufs[s * 3 + t]
            else:
                src = out_ref.at[pl.ds(send_off, rows), pl.ds(c0, cw)]
                dst = out_ref.at[pl.ds(send_off, rows), pl.ds(c0, cw)]
            rdma = pltpu.make_async_remote_copy(
                src_ref=src,
                dst_ref=dst,
                send_sem=send_sems.at[s],
                recv_sem=recv_sems.at[s * 6 + t],
                device_id=(pid,),
                device_id_type=pl.DeviceIdType.MESH,
            )
            rdma.start()
            return rdma

        def complete(s, t, rdma):
            phase, rows, _, _, recv_off = plans[s][t]
            c0, c1 = SLAB_COLS[s]
            rdma.wait()
            if phase == "rs":
                acc_bf[pl.ds(recv_off, rows), c0:c1] += bufs[s * 3 + t][...]
                if t == 2:
                    out_ref[pl.ds(recv_off, 128), c0:c1] = \
                        acc_bf[pl.ds(recv_off, 128), c0:c1]

        inflight = [None, None, None]
        for s in range(3):
            sub_dot(s, plans[s][0][3])
            inflight[s] = issue(s, 0)
        for s in range(3):
            sub_dot(s, plans[s][0][4])

        for t in range(1, 6):
            for s in range(3):
                complete(s, t - 1, inflight[s])
                inflight[s] = issue(s, t)
        for s in range(3):
            complete(s, 5, inflight[s])

    scratch = [
        pltpu.VMEM((M, N), jnp.bfloat16),
        pltpu.VMEM((2, M, K // KB), jnp.float32),
        pltpu.VMEM((2, N, K // KB), jnp.float32),
        pltpu.VMEM((M, K), jnp.bfloat16),
        pltpu.VMEM((N, K), jnp.bfloat16),
    ]
    for s in range(3):
        cw = SLAB_COLS[s][1] - SLAB_COLS[s][0]
        for rows in (512, 256, 128):
            scratch.append(pltpu.VMEM((rows, cw), jnp.bfloat16))
    scratch.append(pltpu.SemaphoreType.DMA((2, 2)))
    scratch.append(pltpu.SemaphoreType.DMA((3,)))
    scratch.append(pltpu.SemaphoreType.DMA((18,)))

    return pl.pallas_call(
        body,
        out_shape=jax.ShapeDtypeStruct((M, N), jnp.bfloat16),
        in_specs=[
            pl.BlockSpec(memory_space=pl.ANY),
            pl.BlockSpec(memory_space=pl.ANY),
        ],
        out_specs=pl.BlockSpec(memory_space=pltpu.VMEM),
        scratch_shapes=scratch,
        compiler_params=pltpu.CompilerParams(collective_id=0),
    )(dy, W)


# device time: 50569 ns/iter; 1.0802x vs baseline; 1.0802x over previous
import jax
import jax.numpy as jnp
from jax import lax
from jax.experimental import pallas as pl
from jax.experimental.pallas import tpu as pltpu

N_DEV = 8
M = 1024
N = 1024
K = 4096
KB = 4

SLAB_COLS = ((0, 384), (384, 768), (768, 1024))
SLAB_AXES = ((0, 1, 2), (1, 2, 0), (2, 0, 1))
SLOTS = (("rs", 512, 2), ("rs", 256, 2), ("x", 256, 2),
         ("ag", 256, 1), ("ag", 512, 1))
SEM_BASE = (0, 2, 4, 6, 7)
SEMS_PER_SLAB = 8


def kernel(dy, W):
    def body(dy_ref, w_ref, out_ref, acc_ref, acc_bf, a_vm, b_vm, *bufs):
        in_sems = bufs[9]
        send_sems = bufs[10]
        recv_sems = bufs[11]

        me = lax.axis_index("i")
        zb = lax.shift_right_logical(me, 2) & 1
        yb = lax.shift_right_logical(me, 1) & 1
        xb = (me ^ lax.shift_right_logical(me, 1)) & 1

        p_z = me ^ 4
        p_y = (me & 4) | ((me & 3) ^ 3)
        p_x = (me & 4) | ((me & 3) ^ 1)
        bit = {0: zb, 1: yb, 2: xb}
        partner = {0: p_z, 1: p_y, 2: p_x}

        kc = K // KB

        def load(kb, slot):
            cpa = pltpu.make_async_copy(
                dy_ref.at[:, kb * kc:(kb + 1) * kc], a_vm.at[slot],
                in_sems.at[slot, 0])
            cpb = pltpu.make_async_copy(
                w_ref.at[:, kb * kc:(kb + 1) * kc], b_vm.at[slot],
                in_sems.at[slot, 1])
            cpa.start()
            cpb.start()
            return cpa, cpb

        pending = load(0, 0)
        for kb in range(KB):
            slot = kb % 2
            cpa, cpb = pending
            if kb + 1 < KB:
                pending = load(kb + 1, 1 - slot)
            cpa.wait()
            cpb.wait()
            a = a_vm[slot].astype(jnp.bfloat16)
            b = b_vm[slot].astype(jnp.bfloat16)
            p = lax.dot_general(
                a, b, (((1,), (1,)), ((), ())),
                preferred_element_type=jnp.float32,
            )
            if kb == 0:
                acc_ref[...] = p
            else:
                acc_ref[...] += p
        acc_bf[...] = acc_ref[...].astype(jnp.bfloat16)

        barrier_sem = pltpu.get_barrier_semaphore()
        for nbr in (p_x, p_y, p_z):
            pl.semaphore_signal(
                barrier_sem, inc=1,
                device_id=(nbr,), device_id_type=pl.DeviceIdType.MESH,
            )
        pl.semaphore_wait(barrier_sem, 3)

        plans = []
        for s in range(3):
            a0, a1, a2 = SLAB_AXES[s]
            b0, b1, b2 = bit[a0], bit[a1], bit[a2]
            p0, p1, p2 = partner[a0], partner[a1], partner[a2]
            my_b1 = b0 * 512
            my_b2 = my_b1 + b1 * 256
            plans.append([
                (p0, (1 - b0) * 512, my_b1),
                (p1, my_b1 + (1 - b1) * 256, my_b2),
                (p2, my_b2, my_b2),
                (p1, my_b2, my_b2),
                (p0, my_b1, my_b1),
            ])

        def issue(s, t):
            kind, rows, halves = SLOTS[t]
            pid, send_off, _ = plans[s][t]
            c0, _ = SLAB_COLS[s]
            cw = SLAB_COLS[s][1] - c0
            hr = rows // halves
            rdmas = []
            for h in range(halves):
                if kind == "ag":
                    src = out_ref.at[pl.ds(send_off + h * hr, hr),
                                     pl.ds(c0, cw)]
                    dst = out_ref.at[pl.ds(send_off + h * hr, hr),
                                     pl.ds(c0, cw)]
                else:
                    src = acc_bf.at[pl.ds(send_off + h * hr, hr),
                                    pl.ds(c0, cw)]
                    dst = bufs[s * 3 + t].at[h * hr:(h + 1) * hr]
                rdma = pltpu.make_async_remote_copy(
                    src_ref=src,
                    dst_ref=dst,
                    send_sem=send_sems.at[s, h],
                    recv_sem=recv_sems.at[s * SEMS_PER_SLAB + SEM_BASE[t] + h],
                    device_id=(pid,),
                    device_id_type=pl.DeviceIdType.MESH,
                )
                rdma.start()
                rdmas.append(rdma)
            return rdmas

        def complete(s, t, rdmas):
            kind, rows, halves = SLOTS[t]
            _, _, recv_off = plans[s][t]
            c0, c1 = SLAB_COLS[s]
            hr = rows // halves
            for h, rdma in enumerate(rdmas):
                rdma.wait()
                if kind != "ag":
                    acc_bf[pl.ds(recv_off + h * hr, hr), c0:c1] += \
                        bufs[s * 3 + t][h * hr:(h + 1) * hr]
            if kind == "x":
                out_ref[pl.ds(recv_off, rows), c0:c1] = \
                    acc_bf[pl.ds(recv_off, rows), c0:c1]

        inflight = [issue(s, 0) for s in range(3)]
        for t in range(1, 5):
            for s in range(3):
                complete(s, t - 1, inflight[s])
                inflight[s] = issue(s, t)
        for s in range(3):
            complete(s, 4, inflight[s])

    scratch = [
        pltpu.VMEM((M, N), jnp.float32),
        pltpu.VMEM((M, N), jnp.bfloat16),
        pltpu.VMEM((2, M, K // KB), jnp.float32),
        pltpu.VMEM((2, N, K // KB), jnp.float32),
    ]
    for s in range(3):
        cw = SLAB_COLS[s][1] - SLAB_COLS[s][0]
        for rows in (512, 256, 256):
            scratch.append(pltpu.VMEM((rows, cw), jnp.bfloat16))
    scratch.append(pltpu.SemaphoreType.DMA((2, 2)))
    scratch.append(pltpu.SemaphoreType.DMA((3, 2)))
    scratch.append(pltpu.SemaphoreType.DMA((24,)))

    return pl.pallas_call(
        body,
        out_shape=jax.ShapeDtypeStruct((M, N), jnp.bfloat16),
        in_specs=[
            pl.BlockSpec(memory_space=pl.ANY),
            pl.BlockSpec(memory_space=pl.ANY),
        ],
        out_specs=pl.BlockSpec(memory_space=pltpu.VMEM),
        scratch_shapes=scratch,
        compiler_params=pltpu.CompilerParams(collective_id=0),
    )(dy, W)
